# baseline (speedup 1.0000x reference)
import functools

import jax
import jax.numpy as jnp
from jax import lax
from jax.experimental import pallas as pl
from jax.experimental.pallas import tpu as pltpu
from jax.experimental.pallas import tpu_sc as plsc

_NC = 2
_NS = 16
_NW = _NC * _NS
_CH = 128
_MB = 2
_NBUF = 5


def _sc_embedding_gather(table, ids4):
    nw, g_blocks, mb, ch = ids4.shape
    d = table.shape[1]
    n_rounds = g_blocks // _NBUF
    assert g_blocks % _NBUF == 0
    mesh = plsc.VectorSubcoreMesh(core_axis_name="c", subcore_axis_name="s")

    @functools.partial(
        pl.kernel,
        mesh=mesh,
        out_type=jax.ShapeDtypeStruct((nw * g_blocks, mb, ch, d), jnp.float32),
        scratch_types=(
            [pltpu.VMEM((g_blocks, mb, ch), jnp.int32)]
            + [pltpu.VMEM((mb, ch, d), jnp.float32) for _ in range(_NBUF)]
            + [pltpu.SemaphoreType.DMA for _ in range(3 * _NBUF)]
        ),
        compiler_params=pltpu.CompilerParams(use_tc_tiling_on_sc=False),
    )
    def k(table_hbm, idx_hbm, out_hbm, idx_v, *scratch):
        bufs = scratch[:_NBUF]
        sem_g = scratch[_NBUF:3 * _NBUF]
        sem_w = scratch[3 * _NBUF:]
        wid = lax.axis_index("s") * _NC + lax.axis_index("c")
        base = wid * g_blocks
        pltpu.sync_copy(idx_hbm.at[wid], idx_v)

        def fire_gathers(slot, c):
            for m in range(mb):
                pltpu.async_copy(
                    table_hbm.at[idx_v.at[c, m]],
                    bufs[slot].at[m],
                    sem_g[mb * slot + m],
                )

        for slot in range(_NBUF):
            fire_gathers(slot, slot)

        def round_body(g, carry):
            cbase = g * _NBUF
            for slot in range(_NBUF):
                for m in range(mb):
                    pltpu.make_async_copy(
                        table_hbm.at[idx_v.at[cbase + slot, m]],
                        bufs[slot].at[m],
                        sem_g[mb * slot + m],
                    ).wait()
                pltpu.async_copy(
                    bufs[slot], out_hbm.at[base + cbase + slot], sem_w[slot]
                )
            for slot in range(_NBUF):
                pltpu.make_async_copy(
                    bufs[slot], out_hbm.at[base + cbase + slot], sem_w[slot]
                ).wait()

                @pl.when(g < n_rounds - 1)
                def _():
                    fire_gathers(slot, cbase + _NBUF + slot)

            return carry

        lax.fori_loop(0, n_rounds, round_body, 0)

    return k(table, ids4)


def kernel(token_ids, embedding_table):
    batch, hist = token_ids.shape
    d = embedding_table.shape[1]
    ids = token_ids.reshape(_NW, -1, _MB, _CH).astype(jnp.int32)
    out = _sc_embedding_gather(embedding_table, ids)
    return out.reshape(batch, hist, d)
